# baseline (speedup 1.0000x reference)
import jax
import jax.numpy as jnp
from jax.experimental import pallas as pl


def _silu(z):
    return z / (1.0 + jnp.exp(-z))


def _mlp_body(p_ref, out_ref):
    x = p_ref[64, 0:16]
    h1 = _silu(jnp.sum(p_ref[0:24, 0:16] * x[None, :], axis=1) + p_ref[65, :])
    h2 = _silu(jnp.sum(p_ref[24:48, :] * h1[None, :], axis=1) + p_ref[66, :])
    out_ref[...] = (jnp.sum(p_ref[48:64, :] * h2[None, :], axis=1)
                    + p_ref[67, 0:16])


def kernel(x, W1, W2, W3, bias):
    packed = jnp.concatenate([
        jnp.pad(W1, ((0, 0), (0, 8))),
        W2,
        W3,
        jnp.pad(x, (0, 8))[None, :],
        bias[16:40][None, :],
        bias[40:64][None, :],
        jnp.pad(bias[64:80], (0, 8))[None, :],
    ], axis=0)
    return pl.pallas_call(
        _mlp_body,
        out_shape=jax.ShapeDtypeStruct((16,), jnp.float32),
    )(packed)


# speedup vs baseline: 2.2693x; 2.2693x over previous
import jax
import jax.numpy as jnp
from jax.experimental import pallas as pl


def _silu(z):
    return z * jax.lax.logistic(z)


def _mlp_body(x_ref, w1_ref, w2_ref, w3_ref, b_ref, out_ref):
    f32 = jnp.float32
    x = x_ref[...]
    b = b_ref[...]
    h1 = _silu(jnp.dot(w1_ref[...], x, preferred_element_type=f32) + b[16:40])
    h2 = _silu(jnp.dot(w2_ref[...], h1, preferred_element_type=f32) + b[40:64])
    y = jnp.dot(w3_ref[...], h2, preferred_element_type=f32) + b[64:80]
    out_ref[...] = y


def kernel(x, W1, W2, W3, bias):
    return pl.pallas_call(
        _mlp_body,
        out_shape=jax.ShapeDtypeStruct((16,), jnp.float32),
    )(x, W1, W2, W3, bias)
